# baseline (speedup 1.0000x reference)
import functools

import jax
import jax.numpy as jnp
from jax import lax
from jax.experimental import pallas as pl
from jax.experimental.pallas import tpu as pltpu
from jax.experimental.pallas import tpu_sc as plsc

_B = 4
_S = 8192
_H = 64
_TOT = _B * _S
_NC = 2
_NS = 16
_NW = _NC * _NS
_PER_W = _TOT // _NW
_CHUNK = 128
_NCH = _PER_W // _CHUNK
_L = 16
_KV = _CHUNK // _L

_mesh = plsc.VectorSubcoreMesh(core_axis_name="c", subcore_axis_name="s")


def _emb_body(x_hbm, p_hbm, wtab_hbm, ptab_hbm, out_hbm,
              xi_v, pi_v, wblk_v, pbuf_v, pidx_v, slab_v, semw, semp):
    wid = lax.axis_index("s") * _NC + lax.axis_index("c")
    b = wid // 8
    s0 = (wid % 8) * _PER_W
    base = wid * _PER_W

    pltpu.sync_copy(x_hbm.at[pl.ds(base, _PER_W)], xi_v)
    pltpu.sync_copy(p_hbm.at[pl.ds(base, _PER_W)], pi_v)

    lane = lax.broadcasted_iota(jnp.int32, (_L,), 0)

    def issue(soff, half):
        copies = []
        for k2 in range(2):
            blkv = xi_v[pl.ds(soff + k2 * _L, _L)] >> 3
            for r in range(_L):
                blk = lax.reduce_max(
                    jnp.where(lane == r, blkv, 0), axes=(0,))
                copies.append(pltpu.async_copy(
                    wtab_hbm.at[blk],
                    wblk_v.at[pl.ds(half * 256 + (k2 * _L + r) * 8, 8), :],
                    semw))
        return copies

    def prep_pos(off):
        for k in range(_KV):
            pv = pi_v[pl.ds(off + k * _L, _L)]
            pidx_v[pl.ds(k * _L, _L)] = pv >> 1
        return pltpu.async_copy(ptab_hbm.at[pidx_v], pbuf_v, semp)

    def assemble(off, sub, half):
        for k2 in range(2):
            k = sub * 2 + k2
            xv = xi_v[pl.ds(off + k * _L, _L)]
            pv = pi_v[pl.ds(off + k * _L, _L)]
            wrow = half * 256 + (k2 * _L + lane) * 8 + (xv & 7)
            prows = lane + k * _L
            pb = (pv & 1) * _H

            zero = jnp.full((_L,), 0, jnp.int32)

            def feat_body(j, cr):
                wv = plsc.load_gather(wblk_v, [cr[0], cr[3]])
                pv2 = plsc.load_gather(pbuf_v, [cr[1], cr[2]])
                slab_v[j, pl.ds(k * _L, _L)] = wv + pv2
                return (cr[0], cr[1], cr[2] + 1, cr[3] + 1)

            lax.fori_loop(0, _H, feat_body, (wrow, prows, pb, zero),
                          unroll=16)

    def write_out(c):
        pltpu.sync_copy(slab_v,
                        out_hbm.at[b, :, pl.ds(s0 + c * _CHUNK, _CHUNK)])

    def pair_body(cc, carry):
        ca = 2 * cc
        offa = ca * _CHUNK
        offb = offa + _CHUNK
        cp = prep_pos(offa)
        pend = {0: issue(offa, 0), 1: issue(offa + 32, 1)}
        cp.wait()
        for sub in range(4):
            half = sub % 2
            for cw in pend[half]:
                cw.wait()
            assemble(offa, sub, half)
            nxt = offa + (sub + 2) * 32
            pend[half] = issue(nxt, half)
        cpb = prep_pos(offb)
        write_out(ca)
        cpb.wait()
        for sub in range(4):
            half = sub % 2
            for cw in pend[half]:
                cw.wait()
            assemble(offb, sub, half)
            if sub < 2:
                pend[half] = issue(offb + (sub + 2) * 32, half)
        write_out(ca + 1)
        return carry

    lax.fori_loop(0, _NCH // 2, pair_body, 0)


_emb = functools.partial(
    pl.kernel,
    out_type=jax.ShapeDtypeStruct((_B, _H, _S), jnp.float32),
    mesh=_mesh,
    scratch_types=[
        pltpu.VMEM((_PER_W,), jnp.int32),
        pltpu.VMEM((_PER_W,), jnp.int32),
        pltpu.VMEM((512, _H), jnp.float32),
        pltpu.VMEM((_CHUNK, 128), jnp.float32),
        pltpu.VMEM((_CHUNK,), jnp.int32),
        pltpu.VMEM((_H, _CHUNK), jnp.float32),
        pltpu.SemaphoreType.DMA,
        pltpu.SemaphoreType.DMA,
    ],
    compiler_params=pltpu.CompilerParams(needs_layout_passes=False),
)(_emb_body)


@jax.jit
def kernel(x, position_ids, word_table, pos_table):
    xf = x.reshape(-1).astype(jnp.int32)
    pf = position_ids.reshape(-1).astype(jnp.int32)
    wt3 = word_table.reshape(125000, 8, _H)
    pt2 = pos_table.reshape(4096, 2 * _H)
    out = _emb(xf, pf, wt3, pt2)
    return out.transpose(0, 2, 1)


# speedup vs baseline: 1.0061x; 1.0061x over previous
import functools

import jax
import jax.numpy as jnp
from jax import lax
from jax.experimental import pallas as pl
from jax.experimental.pallas import tpu as pltpu
from jax.experimental.pallas import tpu_sc as plsc

_B = 4
_S = 8192
_H = 64
_TOT = _B * _S
_NC = 2
_NS = 16
_NW = _NC * _NS
_PER_W = _TOT // _NW
_CHUNK = 128
_NCH = _PER_W // _CHUNK
_L = 16
_KV = _CHUNK // _L

_mesh = plsc.VectorSubcoreMesh(core_axis_name="c", subcore_axis_name="s")


def _emb_body(x_hbm, p_hbm, wtab_hbm, ptab_hbm, out_hbm,
              xi_v, pi_v, wblk_v, pbuf_v, pidx_v, slab_v, semw, semp):
    wid = lax.axis_index("s") * _NC + lax.axis_index("c")
    b = wid // 8
    s0 = (wid % 8) * _PER_W
    base = wid * _PER_W

    pltpu.sync_copy(x_hbm.at[pl.ds(base, _PER_W)], xi_v)
    pltpu.sync_copy(p_hbm.at[pl.ds(base, _PER_W)], pi_v)

    lane = lax.broadcasted_iota(jnp.int32, (_L,), 0)

    def issue(soff, half):
        copies = []
        for k2 in range(2):
            blkv = xi_v[pl.ds(soff + k2 * _L, _L)] >> 3
            for r in range(_L):
                blk = lax.reduce_max(
                    jnp.where(lane == r, blkv, 0), axes=(0,))
                copies.append(pltpu.async_copy(
                    wtab_hbm.at[blk],
                    wblk_v.at[pl.ds(half * 256 + (k2 * _L + r) * 8, 8), :],
                    semw))
        return copies

    def prep_pos(off):
        for k in range(_KV):
            pv = pi_v[pl.ds(off + k * _L, _L)]
            pidx_v[pl.ds(k * _L, _L)] = pv >> 1
        return pltpu.async_copy(ptab_hbm.at[pidx_v], pbuf_v, semp)

    def assemble(off, sub, half):
        for k2 in range(2):
            k = sub * 2 + k2
            xv = xi_v[pl.ds(off + k * _L, _L)]
            pv = pi_v[pl.ds(off + k * _L, _L)]
            wrow = half * 256 + (k2 * _L + lane) * 8 + (xv & 7)
            prows = lane + k * _L
            pb = (pv & 1) * _H

            zero = jnp.full((_L,), 0, jnp.int32)

            def feat_body(j, cr):
                wv = plsc.load_gather(wblk_v, [cr[0], cr[3]])
                pv2 = plsc.load_gather(pbuf_v, [cr[1], cr[2]])
                slab_v[j, pl.ds(k * _L, _L)] = wv + pv2
                return (cr[0], cr[1], cr[2] + 1, cr[3] + 1)

            lax.fori_loop(0, _H, feat_body, (wrow, prows, pb, zero),
                          unroll=8)

    def write_out(c):
        pltpu.sync_copy(slab_v,
                        out_hbm.at[b, :, pl.ds(s0 + c * _CHUNK, _CHUNK)])

    def pair_body(cc, carry):
        ca = 2 * cc
        offa = ca * _CHUNK
        offb = offa + _CHUNK
        cp = prep_pos(offa)
        pend = {0: issue(offa, 0), 1: issue(offa + 32, 1)}
        cp.wait()
        for sub in range(4):
            half = sub % 2
            for cw in pend[half]:
                cw.wait()
            assemble(offa, sub, half)
            nxt = offa + (sub + 2) * 32
            pend[half] = issue(nxt, half)
        cpb = prep_pos(offb)
        write_out(ca)
        cpb.wait()
        for sub in range(4):
            half = sub % 2
            for cw in pend[half]:
                cw.wait()
            assemble(offb, sub, half)
            if sub < 2:
                pend[half] = issue(offb + (sub + 2) * 32, half)
        write_out(ca + 1)
        return carry

    lax.fori_loop(0, _NCH // 2, pair_body, 0)


_emb = functools.partial(
    pl.kernel,
    out_type=jax.ShapeDtypeStruct((_B, _H, _S), jnp.float32),
    mesh=_mesh,
    scratch_types=[
        pltpu.VMEM((_PER_W,), jnp.int32),
        pltpu.VMEM((_PER_W,), jnp.int32),
        pltpu.VMEM((512, _H), jnp.float32),
        pltpu.VMEM((_CHUNK, 128), jnp.float32),
        pltpu.VMEM((_CHUNK,), jnp.int32),
        pltpu.VMEM((_H, _CHUNK), jnp.float32),
        pltpu.SemaphoreType.DMA,
        pltpu.SemaphoreType.DMA,
    ],
    compiler_params=pltpu.CompilerParams(needs_layout_passes=False),
)(_emb_body)


@jax.jit
def kernel(x, position_ids, word_table, pos_table):
    xf = x.reshape(-1).astype(jnp.int32)
    pf = position_ids.reshape(-1).astype(jnp.int32)
    wt3 = word_table.reshape(125000, 8, _H)
    pt2 = pos_table.reshape(4096, 2 * _H)
    out = _emb(xf, pf, wt3, pt2)
    return out.transpose(0, 2, 1)
